# baseline (speedup 1.0000x reference)
import jax
import jax.numpy as jnp
from jax.experimental import pallas as pl
from jax.experimental.pallas import tpu as pltpu

_BM = 200
_RING = 5


def _make_body(n_rows, n_cols, n_steps, ring):
    def _body(adj_hbm, x_hbm, wt_ref, b_ref, out_ref,
              a_ring, x_res, a_sems, x_sem):
        i = pl.program_id(0)
        slot = jax.lax.rem(i, ring)

        @pl.when(i == 0)
        def _prime():
            pltpu.make_async_copy(x_hbm, x_res, x_sem).start()
            for s in range(ring):
                pltpu.make_async_copy(
                    adj_hbm.at[pl.ds(s * _BM, _BM), :],
                    a_ring.at[s], a_sems.at[s]).start()
            pltpu.make_async_copy(x_hbm, x_res, x_sem).wait()

        pltpu.make_async_copy(
            adj_hbm.at[pl.ds(i * _BM, _BM), :],
            a_ring.at[slot], a_sems.at[slot]).wait()

        h = jnp.dot(a_ring[slot], x_res[...],
                    preferred_element_type=jnp.float32)
        o = jnp.dot(h, wt_ref[...], preferred_element_type=jnp.float32)
        out_ref[...] = o + b_ref[...]

        nxt = i + ring

        @pl.when(nxt < n_steps)
        def _prefetch():
            pltpu.make_async_copy(
                adj_hbm.at[pl.ds(nxt * _BM, _BM), :],
                a_ring.at[slot], a_sems.at[slot]).start()

    return _body


def kernel(x, adj, W, b):
    n_rows, n_cols = adj.shape
    d_in = x.shape[1]
    d_out = W.shape[0]

    wt = W.T
    b2 = b.reshape(1, d_out)

    n_steps = n_rows // _BM
    ring = min(_RING, n_steps)

    return pl.pallas_call(
        _make_body(n_rows, n_cols, n_steps, ring),
        grid=(n_steps,),
        in_specs=[
            pl.BlockSpec(memory_space=pltpu.MemorySpace.HBM),
            pl.BlockSpec(memory_space=pltpu.MemorySpace.HBM),
            pl.BlockSpec((d_in, d_out), lambda i: (0, 0)),
            pl.BlockSpec((1, d_out), lambda i: (0, 0)),
        ],
        out_specs=pl.BlockSpec((_BM, d_out), lambda i: (i, 0)),
        out_shape=jax.ShapeDtypeStruct((n_rows, d_out), jnp.float32),
        scratch_shapes=[
            pltpu.VMEM((ring, _BM, n_cols), jnp.float32),
            pltpu.VMEM((n_cols, d_in), jnp.float32),
            pltpu.SemaphoreType.DMA((ring,)),
            pltpu.SemaphoreType.DMA,
        ],
        compiler_params=pltpu.CompilerParams(
            dimension_semantics=("arbitrary",),
            vmem_limit_bytes=64 * 1024 * 1024,
        ),
    )(adj, x, wt, b2)
